# baseline (speedup 1.0000x reference)
import functools

import jax
import jax.numpy as jnp
from jax import lax
from jax.experimental import pallas as pl
from jax.experimental.pallas import tpu as pltpu
from jax.experimental.pallas import tpu_sc as plsc

_LANES = 16
_NC = 2
_NS = 16
_NW = _NC * _NS
_CHUNK = 128


def _make_sc_reverse(maxlen, dim, dtype):
    rows_per_w = maxlen // _NW
    n_chunks = rows_per_w // _CHUNK
    mesh = plsc.VectorSubcoreMesh(core_axis_name="c", subcore_axis_name="s")

    @functools.partial(
        pl.kernel,
        mesh=mesh,
        out_type=jax.ShapeDtypeStruct((maxlen, dim), dtype),
        scratch_types=[
            pltpu.VMEM((_CHUNK,), jnp.int32),
            pltpu.VMEM((_CHUNK, dim), dtype),
            pltpu.SemaphoreType.DMA,
        ],
    )
    def rev(table_hbm, out_hbm, idx_v, rows_v, sem):
        wid = lax.axis_index("s") * _NC + lax.axis_index("c")
        base = wid * rows_per_w
        lane = lax.broadcasted_iota(jnp.int32, (_LANES,), 0)
        for ch in range(n_chunks):
            top = maxlen - 1 - base - ch * _CHUNK
            for i in range(_CHUNK // _LANES):
                idx_v[pl.ds(i * _LANES, _LANES)] = (top - i * _LANES) - lane
            pltpu.async_copy(table_hbm.at[idx_v], rows_v, sem).wait()
            pltpu.sync_copy(rows_v, out_hbm.at[pl.ds(base + ch * _CHUNK, _CHUNK)])

    return rev


def kernel(x, pos_emb):
    maxlen = x.shape[-1]
    dim = pos_emb.shape[1]
    rev = _make_sc_reverse(maxlen, dim, pos_emb.dtype)
    return rev(pos_emb[:maxlen])


# speedup vs baseline: 1.0312x; 1.0312x over previous
import functools

import jax
import jax.numpy as jnp
from jax import lax
from jax.experimental import pallas as pl
from jax.experimental.pallas import tpu as pltpu
from jax.experimental.pallas import tpu_sc as plsc

_LANES = 16
_NC = 2
_NS = 16
_NW = _NC * _NS
_CHUNK = 128


def _make_sc_reverse(maxlen, dim, dtype):
    rows_per_w = maxlen // _NW
    n_chunks = rows_per_w // _CHUNK
    mesh = plsc.VectorSubcoreMesh(core_axis_name="c", subcore_axis_name="s")

    @functools.partial(
        pl.kernel,
        mesh=mesh,
        out_type=jax.ShapeDtypeStruct((maxlen, dim), dtype),
        scratch_types=(
            [pltpu.VMEM((_CHUNK,), jnp.int32) for _ in range(n_chunks)]
            + [pltpu.VMEM((_CHUNK, dim), dtype) for _ in range(n_chunks)]
            + [pltpu.SemaphoreType.DMA for _ in range(n_chunks)]
        ),
    )
    def rev(table_hbm, out_hbm, *scratch):
        idx_refs = scratch[:n_chunks]
        row_refs = scratch[n_chunks:2 * n_chunks]
        sems = scratch[2 * n_chunks:]
        wid = lax.axis_index("s") * _NC + lax.axis_index("c")
        base = wid * rows_per_w
        lane = lax.broadcasted_iota(jnp.int32, (_LANES,), 0)
        copies = []
        for ch in range(n_chunks):
            top = maxlen - 1 - base - ch * _CHUNK
            for i in range(_CHUNK // _LANES):
                idx_refs[ch][pl.ds(i * _LANES, _LANES)] = (top - i * _LANES) - lane
            copies.append(
                pltpu.async_copy(table_hbm.at[idx_refs[ch]], row_refs[ch], sems[ch]))
        for ch in range(n_chunks):
            copies[ch].wait()
            pltpu.sync_copy(
                row_refs[ch], out_hbm.at[pl.ds(base + ch * _CHUNK, _CHUNK)])

    return rev


def kernel(x, pos_emb):
    maxlen = x.shape[-1]
    dim = pos_emb.shape[1]
    rev = _make_sc_reverse(maxlen, dim, pos_emb.dtype)
    return rev(pos_emb[:maxlen])
